# Initial kernel scaffold; baseline (speedup 1.0000x reference)
import jax
import jax.numpy as jnp
from jax.experimental import pallas as pl


def kernel(boxes, scores):
    raise NotImplementedError("write your pallas kernel here")


# speedup vs baseline: 5.7499x; 5.7499x over previous
import functools
import jax
import jax.numpy as jnp
from jax import lax
from jax.experimental import pallas as pl
from jax.experimental.pallas import tpu as pltpu
from jax.experimental.pallas import tpu_sc as plsc

_IOU_THRESHOLD = 0.5
_MAX_OUT = 200
_N = 5000
_LANES = 16
_SUBCORES = 16
_NP = 5120
_PER = _NP // _SUBCORES
_VPT = _PER // _LANES

_NEG = float("-inf")
_BIG = 2**30


def _nms_body(bx_h, by_h, bw_h, bh_h, sc_h, out_h,
              bxv, byv, bwv, bhv,
              x1f, x3f, y1f, y3f, arf,
              msc, cur, sel, outv,
              pubv, pubi, allv, alli,
              shv, shi):
    c = lax.axis_index("c")
    s = lax.axis_index("s")
    base = s * _PER

    pltpu.sync_copy(bx_h, bxv)
    pltpu.sync_copy(by_h, byv)
    pltpu.sync_copy(bw_h, bwv)
    pltpu.sync_copy(bh_h, bhv)
    pltpu.sync_copy(sc_h.at[pl.ds(base, _PER)], msc)

    def initj(j, _):
        sl = pl.ds(j * _LANES, _LANES)
        x = bxv[sl]
        y = byv[sl]
        w = bwv[sl]
        h = bhv[sl]
        x1f[sl] = x - 0.5 * w
        x3f[sl] = x + 0.5 * w
        y1f[sl] = y - 0.5 * h
        y3f[sl] = y + 0.5 * h
        arf[sl] = w * h
        return 0

    lax.fori_loop(0, _NP // _LANES, initj, 0)

    zeros16 = jnp.zeros((_LANES,), jnp.float32)
    for j in range(_VPT):
        sl = pl.ds(j * _LANES, _LANES)
        cur[sl] = msc[sl]
        sel[sl] = zeros16

    iota16 = lax.iota(jnp.int32, _LANES)

    def xmax16(v):
        for sh in (8, 4, 2, 1):
            p = v.at[iota16 ^ sh].get(mode="promise_in_bounds")
            v = jnp.maximum(v, p)
        return v

    def xmin16i(v):
        for sh in (8, 4, 2, 1):
            p = v.at[iota16 ^ sh].get(mode="promise_in_bounds")
            v = jnp.minimum(v, p)
        return v

    bigv = jnp.full((_LANES,), _BIG, jnp.int32)
    negv = jnp.full((_LANES,), _NEG, jnp.float32)
    ones16 = jnp.full((_LANES,), 1.0, jnp.float32)
    izeros16 = jnp.zeros((_LANES,), jnp.int32)

    def round_body(i, _):
        m = cur[pl.ds(0, _LANES)]
        for j in range(1, _VPT):
            m = jnp.maximum(m, cur[pl.ds(j * _LANES, _LANES)])
        msp = xmax16(m)
        idx = bigv
        for j in range(_VPT):
            v = cur[pl.ds(j * _LANES, _LANES)]
            gio = iota16 + jnp.full((_LANES,), base + j * _LANES, jnp.int32)
            idx = jnp.minimum(idx, jnp.where(v == msp, gio, bigv))
        lisp = xmin16i(idx)

        pubv[...] = msp
        pubi[...] = lisp
        srow = c * (_SUBCORES * _LANES) + s * _LANES
        pltpu.sync_copy(pubv, shv.at[pl.ds(srow, _LANES)])
        pltpu.sync_copy(pubi, shi.at[pl.ds(srow, _LANES)])
        plsc.subcore_barrier()
        pltpu.sync_copy(shv.at[pl.ds(c * (_SUBCORES * _LANES), _SUBCORES * _LANES)], allv)
        pltpu.sync_copy(shi.at[pl.ds(c * (_SUBCORES * _LANES), _SUBCORES * _LANES)], alli)
        plsc.subcore_barrier()

        gm = allv[pl.ds(0, _LANES)]
        for t in range(1, _SUBCORES):
            gm = jnp.maximum(gm, allv[pl.ds(t * _LANES, _LANES)])
        gidx = bigv
        for t in range(_SUBCORES):
            tv = allv[pl.ds(t * _LANES, _LANES)]
            ti = alli[pl.ds(t * _LANES, _LANES)]
            gidx = jnp.minimum(gidx, jnp.where(tv == gm, ti, bigv))
        valid = gm > negv
        safe_idx = jnp.where(valid, gidx, izeros16)

        gx1 = plsc.load_gather(x1f, [safe_idx])
        gx3 = plsc.load_gather(x3f, [safe_idx])
        gy1 = plsc.load_gather(y1f, [safe_idx])
        gy3 = plsc.load_gather(y3f, [safe_idx])
        gar = plsc.load_gather(arf, [safe_idx])

        for j in range(_VPT):
            fsl = pl.ds(base + j * _LANES, _LANES)
            csl = pl.ds(j * _LANES, _LANES)
            xx1 = jnp.maximum(x1f[fsl], gx1)
            xx3 = jnp.minimum(x3f[fsl], gx3)
            yy1 = jnp.maximum(y1f[fsl], gy1)
            yy3 = jnp.minimum(y3f[fsl], gy3)
            inter = jnp.maximum(xx3 - xx1, 0.0) * jnp.maximum(yy3 - yy1, 0.0)
            iou = inter / (arf[fsl] + gar - inter)
            supp = (iou > _IOU_THRESHOLD) & valid
            cur[csl] = jnp.where(supp, negv, cur[csl])
            gio = iota16 + jnp.full((_LANES,), base + j * _LANES, jnp.int32)
            hit = (gio == gidx) & valid
            sel[csl] = sel[csl] + jnp.where(hit, ones16, zeros16)
        return 0

    lax.fori_loop(0, _MAX_OUT, round_body, 0)

    for j in range(_VPT):
        sl = pl.ds(j * _LANES, _LANES)
        outv[sl] = msc[sl] * sel[sl]

    @pl.when(c == 0)
    def _():
        pltpu.sync_copy(outv, out_h.at[pl.ds(base, _PER)])


@jax.jit
def _nms_sc(bx, by, bw, bh, sc):
    mesh = plsc.VectorSubcoreMesh(core_axis_name="c", subcore_axis_name="s")
    f = functools.partial(
        pl.kernel,
        mesh=mesh,
        compiler_params=pltpu.CompilerParams(needs_layout_passes=False),
        out_type=jax.ShapeDtypeStruct((_NP,), jnp.float32),
        scratch_types=[
            pltpu.VMEM((_NP,), jnp.float32),
            pltpu.VMEM((_NP,), jnp.float32),
            pltpu.VMEM((_NP,), jnp.float32),
            pltpu.VMEM((_NP,), jnp.float32),
            pltpu.VMEM((_NP,), jnp.float32),
            pltpu.VMEM((_NP,), jnp.float32),
            pltpu.VMEM((_NP,), jnp.float32),
            pltpu.VMEM((_NP,), jnp.float32),
            pltpu.VMEM((_NP,), jnp.float32),
            pltpu.VMEM((_PER,), jnp.float32),
            pltpu.VMEM((_PER,), jnp.float32),
            pltpu.VMEM((_PER,), jnp.float32),
            pltpu.VMEM((_PER,), jnp.float32),
            pltpu.VMEM((_LANES,), jnp.float32),
            pltpu.VMEM((_LANES,), jnp.int32),
            pltpu.VMEM((_SUBCORES * _LANES,), jnp.float32),
            pltpu.VMEM((_SUBCORES * _LANES,), jnp.int32),
            pltpu.VMEM_SHARED((2 * _SUBCORES * _LANES,), jnp.float32),
            pltpu.VMEM_SHARED((2 * _SUBCORES * _LANES,), jnp.int32),
        ],
    )(_nms_body)
    return f(bx, by, bw, bh, sc)


def kernel(boxes, scores):
    pad = _NP - _N
    bx = jnp.concatenate([boxes[:, 0], jnp.zeros((pad,), jnp.float32)])
    by = jnp.concatenate([boxes[:, 1], jnp.zeros((pad,), jnp.float32)])
    bw = jnp.concatenate([boxes[:, 2], jnp.zeros((pad,), jnp.float32)])
    bh = jnp.concatenate([boxes[:, 3], jnp.zeros((pad,), jnp.float32)])
    sc = jnp.concatenate([scores, jnp.full((pad,), _NEG, jnp.float32)])
    out = _nms_sc(bx, by, bw, bh, sc)
    return out[:_N]


# speedup vs baseline: 7.0232x; 1.2214x over previous
import functools
import jax
import jax.numpy as jnp
from jax import lax
from jax.experimental import pallas as pl
from jax.experimental.pallas import tpu as pltpu
from jax.experimental.pallas import tpu_sc as plsc

_IOU_THRESHOLD = 0.5
_MAX_OUT = 200
_N = 5000
_LANES = 16
_SUBCORES = 16
_NP = 5120
_PER = _NP // _SUBCORES
_VPT = _PER // _LANES
_ROW = 2 * _LANES
_PLANE = _SUBCORES * _ROW
_SH = 2 * 2 * _PLANE

_NEG = float("-inf")


def _nms_body(bx_h, by_h, bw_h, bh_h, sc_h, out_h,
              bxv, byv, bwv, bhv,
              x1f, x3f, y1f, y3f, arf,
              msc, cur, sel, outv,
              pub, allb, sh):
    c = lax.axis_index("c")
    s = lax.axis_index("s")
    base = s * _PER

    pltpu.sync_copy(bx_h, bxv)
    pltpu.sync_copy(by_h, byv)
    pltpu.sync_copy(bw_h, bwv)
    pltpu.sync_copy(bh_h, bhv)
    pltpu.sync_copy(sc_h.at[pl.ds(base, _PER)], msc)

    def initj(j, _):
        sl = pl.ds(j * _LANES, _LANES)
        x = bxv[sl]
        y = byv[sl]
        w = bwv[sl]
        h = bhv[sl]
        x1f[sl] = x - 0.5 * w
        x3f[sl] = x + 0.5 * w
        y1f[sl] = y - 0.5 * h
        y3f[sl] = y + 0.5 * h
        arf[sl] = w * h
        return 0

    lax.fori_loop(0, _NP // _LANES, initj, 0)

    zeros16 = jnp.zeros((_LANES,), jnp.float32)
    for j in range(_VPT):
        sl = pl.ds(j * _LANES, _LANES)
        cur[sl] = msc[sl]
        sel[sl] = zeros16

    iota16 = lax.iota(jnp.int32, _LANES)
    negv = jnp.full((_LANES,), _NEG, jnp.float32)
    ones16 = jnp.full((_LANES,), 1.0, jnp.float32)
    izeros16 = jnp.zeros((_LANES,), jnp.int32)
    basev = jnp.full((_LANES,), base, jnp.int32)
    perv = jnp.full((_LANES,), _PER, jnp.int32)
    lane0 = iota16 == izeros16

    def combine(bv, bi, ov, oi):
        take = (ov > bv) | ((ov == bv) & (oi < bi))
        return jnp.where(take, ov, bv), jnp.where(take, oi, bi)

    def round_body(i, _):
        bv = cur[pl.ds(0, _LANES)]
        bi = iota16 + basev
        for j in range(1, _VPT):
            v = cur[pl.ds(j * _LANES, _LANES)]
            gio = iota16 + jnp.full((_LANES,), j * _LANES, jnp.int32) + basev
            better = v > bv
            bv = jnp.where(better, v, bv)
            bi = jnp.where(better, gio, bi)
        for shf in (8, 4, 2, 1):
            perm = iota16 ^ shf
            ov = bv.at[perm].get(mode="promise_in_bounds")
            oi = bi.at[perm].get(mode="promise_in_bounds")
            bv, bi = combine(bv, bi, ov, oi)

        pub[pl.ds(0, _LANES)] = bv
        pub[pl.ds(_LANES, _LANES)] = plsc.bitcast(bi, jnp.float32)
        par = lax.rem(i, 2)
        plane = (par * 2 + c) * _PLANE
        pltpu.sync_copy(pub, sh.at[pl.ds(plane + s * _ROW, _ROW)])
        plsc.subcore_barrier()
        pltpu.sync_copy(sh.at[pl.ds(plane, _PLANE)], allb)

        gv = allb[pl.ds(0, _LANES)]
        gi = plsc.bitcast(allb[pl.ds(_LANES, _LANES)], jnp.int32)
        for t in range(1, _SUBCORES):
            tv = allb[pl.ds(t * _ROW, _LANES)]
            ti = plsc.bitcast(allb[pl.ds(t * _ROW + _LANES, _LANES)], jnp.int32)
            gv, gi = combine(gv, gi, tv, ti)
        valid = gv > negv
        safe_idx = jnp.where(valid, gi, izeros16)

        gx1 = plsc.load_gather(x1f, [safe_idx])
        gx3 = plsc.load_gather(x3f, [safe_idx])
        gy1 = plsc.load_gather(y1f, [safe_idx])
        gy3 = plsc.load_gather(y3f, [safe_idx])
        gar = plsc.load_gather(arf, [safe_idx])

        lidx = gi - basev
        mine = valid & (lidx >= izeros16) & (lidx < perv) & lane0
        plsc.addupdate_scatter(sel, [jnp.where(mine, lidx, izeros16)], ones16, mask=mine)

        for j in range(_VPT):
            fsl = pl.ds(base + j * _LANES, _LANES)
            csl = pl.ds(j * _LANES, _LANES)
            xx1 = jnp.maximum(x1f[fsl], gx1)
            xx3 = jnp.minimum(x3f[fsl], gx3)
            yy1 = jnp.maximum(y1f[fsl], gy1)
            yy3 = jnp.minimum(y3f[fsl], gy3)
            inter = jnp.maximum(xx3 - xx1, 0.0) * jnp.maximum(yy3 - yy1, 0.0)
            iou = inter / (arf[fsl] + gar - inter)
            supp = (iou > _IOU_THRESHOLD) & valid
            cur[csl] = jnp.where(supp, negv, cur[csl])
        return 0

    lax.fori_loop(0, _MAX_OUT, round_body, 0)

    for j in range(_VPT):
        sl = pl.ds(j * _LANES, _LANES)
        outv[sl] = msc[sl] * sel[sl]

    @pl.when(c == 0)
    def _():
        pltpu.sync_copy(outv, out_h.at[pl.ds(base, _PER)])


@jax.jit
def _nms_sc(bx, by, bw, bh, sc):
    mesh = plsc.VectorSubcoreMesh(core_axis_name="c", subcore_axis_name="s")
    f = functools.partial(
        pl.kernel,
        mesh=mesh,
        compiler_params=pltpu.CompilerParams(needs_layout_passes=False),
        out_type=jax.ShapeDtypeStruct((_NP,), jnp.float32),
        scratch_types=[
            pltpu.VMEM((_NP,), jnp.float32),
            pltpu.VMEM((_NP,), jnp.float32),
            pltpu.VMEM((_NP,), jnp.float32),
            pltpu.VMEM((_NP,), jnp.float32),
            pltpu.VMEM((_NP,), jnp.float32),
            pltpu.VMEM((_NP,), jnp.float32),
            pltpu.VMEM((_NP,), jnp.float32),
            pltpu.VMEM((_NP,), jnp.float32),
            pltpu.VMEM((_NP,), jnp.float32),
            pltpu.VMEM((_PER,), jnp.float32),
            pltpu.VMEM((_PER,), jnp.float32),
            pltpu.VMEM((_PER,), jnp.float32),
            pltpu.VMEM((_PER,), jnp.float32),
            pltpu.VMEM((_ROW,), jnp.float32),
            pltpu.VMEM((_PLANE,), jnp.float32),
            pltpu.VMEM_SHARED((_SH,), jnp.float32),
        ],
    )(_nms_body)
    return f(bx, by, bw, bh, sc)


def kernel(boxes, scores):
    pad = _NP - _N
    bx = jnp.concatenate([boxes[:, 0], jnp.zeros((pad,), jnp.float32)])
    by = jnp.concatenate([boxes[:, 1], jnp.zeros((pad,), jnp.float32)])
    bw = jnp.concatenate([boxes[:, 2], jnp.zeros((pad,), jnp.float32)])
    bh = jnp.concatenate([boxes[:, 3], jnp.zeros((pad,), jnp.float32)])
    sc = jnp.concatenate([scores, jnp.full((pad,), _NEG, jnp.float32)])
    out = _nms_sc(bx, by, bw, bh, sc)
    return out[:_N]


# speedup vs baseline: 8.6582x; 1.2328x over previous
import functools
import jax
import jax.numpy as jnp
from jax import lax
from jax.experimental import pallas as pl
from jax.experimental.pallas import tpu as pltpu
from jax.experimental.pallas import tpu_sc as plsc

_IOU_THRESHOLD = 0.5
_MAX_OUT = 200
_N = 5000
_LANES = 16
_SUBCORES = 16
_NP = 5120
_PER = _NP // _SUBCORES
_VPT = _PER // _LANES
_ROW = 8
_PLANE = _SUBCORES * _ROW
_SH = 2 * 2 * _PLANE

_NEG = float("-inf")


def _nms_body(bx_h, by_h, bw_h, bh_h, sc_h, out_h,
              bxv, byv, bwv, bhv,
              x1f, x3f, y1f, y3f, arf,
              msc, cur, sel, outv,
              pub, allb, sh):
    c = lax.axis_index("c")
    s = lax.axis_index("s")
    base = s * _PER

    pltpu.sync_copy(bx_h, bxv)
    pltpu.sync_copy(by_h, byv)
    pltpu.sync_copy(bw_h, bwv)
    pltpu.sync_copy(bh_h, bhv)
    pltpu.sync_copy(sc_h.at[pl.ds(base, _PER)], msc)

    def initj(j, _):
        sl = pl.ds(j * _LANES, _LANES)
        x = bxv[sl]
        y = byv[sl]
        w = bwv[sl]
        h = bhv[sl]
        x1f[sl] = x - 0.5 * w
        x3f[sl] = x + 0.5 * w
        y1f[sl] = y - 0.5 * h
        y3f[sl] = y + 0.5 * h
        arf[sl] = w * h
        return 0

    lax.fori_loop(0, _NP // _LANES, initj, 0)

    zeros16 = jnp.zeros((_LANES,), jnp.float32)
    for j in range(_VPT):
        sl = pl.ds(j * _LANES, _LANES)
        cur[sl] = msc[sl]
        sel[sl] = zeros16

    iota16 = lax.iota(jnp.int32, _LANES)
    negv = jnp.full((_LANES,), _NEG, jnp.float32)
    ones16 = jnp.full((_LANES,), 1.0, jnp.float32)
    izeros16 = jnp.zeros((_LANES,), jnp.int32)
    basev = jnp.full((_LANES,), base, jnp.int32)
    perv = jnp.full((_LANES,), _PER, jnp.int32)
    lane0 = iota16 == izeros16
    lane1 = iota16 == jnp.full((_LANES,), 1, jnp.int32)

    def combine(bv, bi, ov, oi):
        take = (ov > bv) | ((ov == bv) & (oi < bi))
        return jnp.where(take, ov, bv), jnp.where(take, oi, bi)

    def round_body(i, carry):
        bv, bi = carry
        for shf in (8, 4, 2, 1):
            perm = iota16 ^ shf
            ov = bv.at[perm].get(mode="promise_in_bounds")
            oi = bi.at[perm].get(mode="promise_in_bounds")
            bv, bi = combine(bv, bi, ov, oi)

        pub[...] = jnp.where(lane1, plsc.bitcast(bi, jnp.float32), bv)
        par = lax.rem(i, 2)
        plane = (par * 2 + c) * _PLANE
        pltpu.sync_copy(pub.at[pl.ds(0, _ROW)], sh.at[pl.ds(plane + s * _ROW, _ROW)])
        plsc.subcore_barrier()
        pltpu.sync_copy(sh.at[pl.ds(plane, _PLANE)], allb)

        gv = plsc.load_gather(allb, [iota16 * _ROW])
        gi = plsc.bitcast(plsc.load_gather(allb, [iota16 * _ROW + jnp.full((_LANES,), 1, jnp.int32)]), jnp.int32)
        for shf in (8, 4, 2, 1):
            perm = iota16 ^ shf
            ov = gv.at[perm].get(mode="promise_in_bounds")
            oi = gi.at[perm].get(mode="promise_in_bounds")
            gv, gi = combine(gv, gi, ov, oi)
        valid = gv > negv
        safe_idx = jnp.where(valid, gi, izeros16)

        gx1 = plsc.load_gather(x1f, [safe_idx])
        gx3 = plsc.load_gather(x3f, [safe_idx])
        gy1 = plsc.load_gather(y1f, [safe_idx])
        gy3 = plsc.load_gather(y3f, [safe_idx])
        gar = plsc.load_gather(arf, [safe_idx])

        lidx = gi - basev
        mine = valid & (lidx >= izeros16) & (lidx < perv) & lane0
        plsc.addupdate_scatter(sel, [jnp.where(mine, lidx, izeros16)], ones16, mask=mine)

        nbv = negv
        nbi = jnp.full((_LANES,), _NP, jnp.int32)
        for j in range(_VPT):
            fsl = pl.ds(base + j * _LANES, _LANES)
            csl = pl.ds(j * _LANES, _LANES)
            xx1 = jnp.maximum(x1f[fsl], gx1)
            xx3 = jnp.minimum(x3f[fsl], gx3)
            yy1 = jnp.maximum(y1f[fsl], gy1)
            yy3 = jnp.minimum(y3f[fsl], gy3)
            inter = jnp.maximum(xx3 - xx1, 0.0) * jnp.maximum(yy3 - yy1, 0.0)
            union = arf[fsl] + gar - inter
            supp = ((33554432.0 * inter - 16777216.0 * union) > union) & valid
            nc = jnp.where(supp, negv, cur[csl])
            cur[csl] = nc
            gio = iota16 + jnp.full((_LANES,), j * _LANES, jnp.int32) + basev
            better = nc > nbv
            nbv = jnp.where(better, nc, nbv)
            nbi = jnp.where(better, gio, nbi)
        return nbv, nbi

    bv0 = cur[pl.ds(0, _LANES)]
    bi0 = iota16 + basev
    for j in range(1, _VPT):
        v = cur[pl.ds(j * _LANES, _LANES)]
        gio0 = iota16 + jnp.full((_LANES,), j * _LANES, jnp.int32) + basev
        better0 = v > bv0
        bv0 = jnp.where(better0, v, bv0)
        bi0 = jnp.where(better0, gio0, bi0)
    lax.fori_loop(0, _MAX_OUT, round_body, (bv0, bi0))

    for j in range(_VPT):
        sl = pl.ds(j * _LANES, _LANES)
        outv[sl] = msc[sl] * sel[sl]

    @pl.when(c == 0)
    def _():
        pltpu.sync_copy(outv, out_h.at[pl.ds(base, _PER)])


@jax.jit
def _nms_sc(bx, by, bw, bh, sc):
    mesh = plsc.VectorSubcoreMesh(core_axis_name="c", subcore_axis_name="s")
    f = functools.partial(
        pl.kernel,
        mesh=mesh,
        compiler_params=pltpu.CompilerParams(needs_layout_passes=False),
        out_type=jax.ShapeDtypeStruct((_NP,), jnp.float32),
        scratch_types=[
            pltpu.VMEM((_NP,), jnp.float32),
            pltpu.VMEM((_NP,), jnp.float32),
            pltpu.VMEM((_NP,), jnp.float32),
            pltpu.VMEM((_NP,), jnp.float32),
            pltpu.VMEM((_NP,), jnp.float32),
            pltpu.VMEM((_NP,), jnp.float32),
            pltpu.VMEM((_NP,), jnp.float32),
            pltpu.VMEM((_NP,), jnp.float32),
            pltpu.VMEM((_NP,), jnp.float32),
            pltpu.VMEM((_PER,), jnp.float32),
            pltpu.VMEM((_PER,), jnp.float32),
            pltpu.VMEM((_PER,), jnp.float32),
            pltpu.VMEM((_PER,), jnp.float32),
            pltpu.VMEM((_LANES,), jnp.float32),
            pltpu.VMEM((_PLANE,), jnp.float32),
            pltpu.VMEM_SHARED((_SH,), jnp.float32),
        ],
    )(_nms_body)
    return f(bx, by, bw, bh, sc)


def kernel(boxes, scores):
    pad = _NP - _N
    bx = jnp.concatenate([boxes[:, 0], jnp.zeros((pad,), jnp.float32)])
    by = jnp.concatenate([boxes[:, 1], jnp.zeros((pad,), jnp.float32)])
    bw = jnp.concatenate([boxes[:, 2], jnp.zeros((pad,), jnp.float32)])
    bh = jnp.concatenate([boxes[:, 3], jnp.zeros((pad,), jnp.float32)])
    sc = jnp.concatenate([scores, jnp.full((pad,), _NEG, jnp.float32)])
    out = _nms_sc(bx, by, bw, bh, sc)
    return out[:_N]


# speedup vs baseline: 12.1084x; 1.3985x over previous
import functools
import jax
import jax.numpy as jnp
from jax import lax
from jax.experimental import pallas as pl
from jax.experimental.pallas import tpu as pltpu
from jax.experimental.pallas import tpu_sc as plsc

_MAX_OUT = 200
_N = 5000
_LANES = 16
_SUBCORES = 16
_NP = 5120
_PER = _NP // _SUBCORES
_VPT = _PER // _LANES
_ROW = 8
_PLANE = _SUBCORES * _ROW
_SH = 2 * 2 * _PLANE

_NEG = float("-inf")


def _nms_body(bx_h, by_h, bw_h, bh_h, sc_h, out_h,
              bxv, byv, bwv, bhv,
              x1f, x3f, y1f, y3f, arf,
              msc, sel, outv,
              pub, allb, sh):
    c = lax.axis_index("c")
    s = lax.axis_index("s")
    base = s * _PER

    pltpu.sync_copy(bx_h, bxv)
    pltpu.sync_copy(by_h, byv)
    pltpu.sync_copy(bw_h, bwv)
    pltpu.sync_copy(bh_h, bhv)
    pltpu.sync_copy(sc_h.at[pl.ds(base, _PER)], msc)

    def initj(j, _):
        sl = pl.ds(j * _LANES, _LANES)
        x = bxv[sl]
        y = byv[sl]
        w = bwv[sl]
        h = bhv[sl]
        x1f[sl] = x - 0.5 * w
        x3f[sl] = x + 0.5 * w
        y1f[sl] = y - 0.5 * h
        y3f[sl] = y + 0.5 * h
        arf[sl] = w * h
        return 0

    lax.fori_loop(0, _NP // _LANES, initj, 0)

    zeros16 = jnp.zeros((_LANES,), jnp.float32)
    for j in range(_VPT):
        sel[pl.ds(j * _LANES, _LANES)] = zeros16

    iota16 = lax.iota(jnp.int32, _LANES)
    negv = jnp.full((_LANES,), _NEG, jnp.float32)
    ones16 = jnp.full((_LANES,), 1.0, jnp.float32)
    izeros16 = jnp.zeros((_LANES,), jnp.int32)
    bigv = jnp.full((_LANES,), 2 ** 30, jnp.int32)
    basev = jnp.full((_LANES,), base, jnp.int32)
    perv = jnp.full((_LANES,), _PER, jnp.int32)
    lane0 = iota16 == izeros16
    lane1 = iota16 == jnp.full((_LANES,), 1, jnp.int32)
    cplane = c * _PLANE

    def xmax(v):
        for shf in (8, 4, 2, 1):
            v = jnp.maximum(v, v.at[iota16 ^ shf].get(mode="promise_in_bounds"))
        return v

    def xmin_i(v):
        for shf in (8, 4, 2, 1):
            v = jnp.minimum(v, v.at[iota16 ^ shf].get(mode="promise_in_bounds"))
        return v

    def one_round(par, bv, bi, curs):
        mx = xmax(bv)
        mi = xmin_i(jnp.where(bv == mx, bi, bigv))

        pub[...] = jnp.where(lane1, plsc.bitcast(mi, jnp.float32), mx)
        plane = cplane + par * 2 * _PLANE
        pltpu.sync_copy(pub.at[pl.ds(0, _ROW)], sh.at[pl.ds(plane + s * _ROW, _ROW)])
        plsc.subcore_barrier()
        pltpu.sync_copy(sh.at[pl.ds(plane, _PLANE)], allb)

        gv = plsc.load_gather(allb, [iota16 * _ROW])
        gi = plsc.bitcast(plsc.load_gather(allb, [iota16 * _ROW + jnp.full((_LANES,), 1, jnp.int32)]), jnp.int32)
        gm = xmax(gv)
        f = plsc.all_reduce_ffs(gv == gm)
        win = gi.at[f].get(mode="promise_in_bounds")
        valid = gm > negv
        safe_idx = jnp.where(valid, win, izeros16)

        gx1 = plsc.load_gather(x1f, [safe_idx])
        gx3 = plsc.load_gather(x3f, [safe_idx])
        gy1 = plsc.load_gather(y1f, [safe_idx])
        gy3 = plsc.load_gather(y3f, [safe_idx])
        gar = plsc.load_gather(arf, [safe_idx])

        lidx = win - basev
        mine = valid & (lidx >= izeros16) & (lidx < perv) & lane0
        plsc.addupdate_scatter(sel, [jnp.where(mine, lidx, izeros16)], ones16, mask=mine)

        nbv = negv
        nbi = bigv
        ncurs = []
        for j in range(_VPT):
            fsl = pl.ds(base + j * _LANES, _LANES)
            xx1 = jnp.maximum(x1f[fsl], gx1)
            xx3 = jnp.minimum(x3f[fsl], gx3)
            yy1 = jnp.maximum(y1f[fsl], gy1)
            yy3 = jnp.minimum(y3f[fsl], gy3)
            inter = jnp.maximum(xx3 - xx1, 0.0) * jnp.maximum(yy3 - yy1, 0.0)
            union = arf[fsl] + gar - inter
            supp = ((33554432.0 * inter - 16777216.0 * union) > union) & valid
            nc = jnp.where(supp, negv, curs[j])
            ncurs.append(nc)
            gio = iota16 + jnp.full((_LANES,), j * _LANES, jnp.int32) + basev
            better = nc > nbv
            nbv = jnp.where(better, nc, nbv)
            nbi = jnp.where(better, gio, nbi)
        return nbv, nbi, ncurs

    def round_pair(i, carry):
        bv, bi = carry[0], carry[1]
        curs = list(carry[2:])
        bv, bi, curs = one_round(0, bv, bi, curs)
        bv, bi, curs = one_round(1, bv, bi, curs)
        return (bv, bi) + tuple(curs)

    curs0 = [msc[pl.ds(j * _LANES, _LANES)] for j in range(_VPT)]
    bv0 = curs0[0]
    bi0 = iota16 + basev
    for j in range(1, _VPT):
        v = curs0[j]
        gio0 = iota16 + jnp.full((_LANES,), j * _LANES, jnp.int32) + basev
        better0 = v > bv0
        bv0 = jnp.where(better0, v, bv0)
        bi0 = jnp.where(better0, gio0, bi0)
    lax.fori_loop(0, _MAX_OUT // 2, round_pair, (bv0, bi0) + tuple(curs0))

    for j in range(_VPT):
        sl = pl.ds(j * _LANES, _LANES)
        outv[sl] = msc[sl] * sel[sl]

    @pl.when(c == 0)
    def _():
        pltpu.sync_copy(outv, out_h.at[pl.ds(base, _PER)])


@jax.jit
def _nms_sc(bx, by, bw, bh, sc):
    mesh = plsc.VectorSubcoreMesh(core_axis_name="c", subcore_axis_name="s")
    f = functools.partial(
        pl.kernel,
        mesh=mesh,
        compiler_params=pltpu.CompilerParams(needs_layout_passes=False),
        out_type=jax.ShapeDtypeStruct((_NP,), jnp.float32),
        scratch_types=[
            pltpu.VMEM((_NP,), jnp.float32),
            pltpu.VMEM((_NP,), jnp.float32),
            pltpu.VMEM((_NP,), jnp.float32),
            pltpu.VMEM((_NP,), jnp.float32),
            pltpu.VMEM((_NP,), jnp.float32),
            pltpu.VMEM((_NP,), jnp.float32),
            pltpu.VMEM((_NP,), jnp.float32),
            pltpu.VMEM((_NP,), jnp.float32),
            pltpu.VMEM((_NP,), jnp.float32),
            pltpu.VMEM((_PER,), jnp.float32),
            pltpu.VMEM((_PER,), jnp.float32),
            pltpu.VMEM((_PER,), jnp.float32),
            pltpu.VMEM((_LANES,), jnp.float32),
            pltpu.VMEM((_PLANE,), jnp.float32),
            pltpu.VMEM_SHARED((_SH,), jnp.float32),
        ],
    )(_nms_body)
    return f(bx, by, bw, bh, sc)


def kernel(boxes, scores):
    pad = _NP - _N
    bx = jnp.concatenate([boxes[:, 0], jnp.zeros((pad,), jnp.float32)])
    by = jnp.concatenate([boxes[:, 1], jnp.zeros((pad,), jnp.float32)])
    bw = jnp.concatenate([boxes[:, 2], jnp.zeros((pad,), jnp.float32)])
    bh = jnp.concatenate([boxes[:, 3], jnp.zeros((pad,), jnp.float32)])
    sc = jnp.concatenate([scores, jnp.full((pad,), _NEG, jnp.float32)])
    out = _nms_sc(bx, by, bw, bh, sc)
    return out[:_N]
